# Initial kernel scaffold; baseline (speedup 1.0000x reference)
import jax
import jax.numpy as jnp
from jax.experimental import pallas as pl


def kernel(x, tok_table, pos_table):
    raise NotImplementedError("write your pallas kernel here")


# speedup vs baseline: 1.2922x; 1.2922x over previous
import functools

import jax
import jax.numpy as jnp
from jax import lax
from jax.experimental import pallas as pl
from jax.experimental.pallas import tpu as pltpu
from jax.experimental.pallas import tpu_sc as plsc

NC = 2
NS = 16
L = 16
NW = NC * NS

CH = 32
NBUF = 3


@functools.partial(jax.jit, static_argnames=("B", "T", "D"))
def _embed(x, tok_table, pos_table, B, T, D):
    TW = T // NW
    NSUB = (B * TW) // CH
    SUB_PER_B = TW // CH

    mesh = plsc.VectorSubcoreMesh(
        core_axis_name="c", subcore_axis_name="s",
        num_cores=NC, num_subcores=NS)

    @functools.partial(
        pl.kernel,
        out_type=jax.ShapeDtypeStruct((B, T, D), jnp.float32),
        mesh=mesh,
        scratch_types=(
            [pltpu.VMEM((B, TW), jnp.int32),
             pltpu.VMEM((TW, D), jnp.float32)]
            + [pltpu.VMEM((CH, D), jnp.float32) for _ in range(NBUF)]
            + [pltpu.SemaphoreType.DMA for _ in range(2 * NBUF + 1)]
        ),
    )
    def run(x_hbm, tok_hbm, pos_hbm, out_hbm, idx_v, pos_v, *rest):
        bufs = list(rest[:NBUF])
        gsems = list(rest[NBUF:2 * NBUF])
        wsems = list(rest[2 * NBUF:3 * NBUF])
        psem = rest[3 * NBUF]

        wid = lax.axis_index("s") * NC + lax.axis_index("c")
        t0 = wid * TW

        for b in range(B):
            pltpu.sync_copy(x_hbm.at[b, pl.ds(t0, TW)], idx_v.at[b])
        pos_cp = pltpu.async_copy(pos_hbm.at[pl.ds(t0, TW)], pos_v, psem)

        gh = [None] * NSUB
        wh = [None] * NSUB

        def issue_gather(i):
            b, h = i // SUB_PER_B, i % SUB_PER_B
            p = i % NBUF
            idx_slice = idx_v.at[b, pl.ds(h * CH, CH)]
            gh[i] = pltpu.async_copy(tok_hbm.at[idx_slice], bufs[p], gsems[p])

        def add_pos(buf, h):
            @plsc.parallel_loop(0, CH, step=1, unroll=1)
            def _row(r):
                for j in range(D // L):
                    s = pl.ds(j * L, L)
                    buf[r, s] = buf[r, s] + pos_v[h * CH + r, s]

        for i in range(min(NBUF - 1, NSUB)):
            issue_gather(i)
        pos_cp.wait()

        for i in range(NSUB):
            b, h = i // SUB_PER_B, i % SUB_PER_B
            p = i % NBUF
            gh[i].wait()
            add_pos(bufs[p], h)
            nxt = i + NBUF - 1
            if nxt < NSUB:
                if i >= 1:
                    wh[i - 1].wait()
                issue_gather(nxt)
            wh[i] = pltpu.async_copy(
                bufs[p], out_hbm.at[b, pl.ds(t0 + h * CH, CH)], wsems[p])

        for i in range(max(0, NSUB - NBUF), NSUB):
            wh[i].wait()

    return run(x, tok_table, pos_table)


def kernel(x, tok_table, pos_table):
    B, T = x.shape
    D = tok_table.shape[1]
    return _embed(x.astype(jnp.int32), tok_table, pos_table, B, T, D)
